# baseline (speedup 1.0000x reference)
import functools

import jax
import jax.numpy as jnp
from jax import lax
from jax.experimental import pallas as pl
from jax.experimental.pallas import tpu as pltpu
from jax.experimental.pallas import tpu_sc as plsc

N = 10000
E = 320000
D = 128
HD = D // 2
G = 64

NC = 2
NS = 16
NW = NC * NS

CH = 128
CPW = 80
EPAD = NW * CPW * CH
WIN = 16
NWIN = CPW // WIN

ACC_ROWS = 10240
ZROWS = 16
ROWS_PER_TILE = ACC_ROWS // NS

BLK = 2000


def _sc_aggregate_body(h2_hbm, src_hbm, dst_hbm, out_hbm,
                       table, acc, src_v, dst_v, rows_v, zbuf, sem0, sem1):
    cid = lax.axis_index("c")
    sid = lax.axis_index("s")
    wid = cid * NS + sid
    base = sid * ROWS_PER_TILE

    zeros16 = jnp.zeros((16,), jnp.float32)

    @pl.loop(0, ZROWS)
    def _zrow(i):
        @pl.loop(0, HD // 16)
        def _zcol(j):
            zbuf[i, pl.ds(j * 16, 16)] = zeros16

    for p in range(2):
        pltpu.sync_copy(h2_hbm.at[p].at[pl.ds(base, ROWS_PER_TILE)],
                        table.at[pl.ds(base, ROWS_PER_TILE)])

        @pl.loop(0, ROWS_PER_TILE // ZROWS)
        def _zacc(r):
            pltpu.sync_copy(zbuf, acc.at[pl.ds(base + r * ZROWS, ZROWS)])

        plsc.subcore_barrier()

        buf0 = rows_v.at[0]
        buf1 = rows_v.at[1]

        @pl.loop(0, NWIN)
        def _win(w):
            pltpu.sync_copy(src_hbm.at[wid].at[pl.ds(w * WIN, WIN)], src_v)
            pltpu.sync_copy(dst_hbm.at[wid].at[pl.ds(w * WIN, WIN)], dst_v)
            pltpu.async_copy(table.at[src_v.at[0]], buf0, sem0)

            @pl.loop(0, WIN // 2)
            def _pair(g):
                j0 = 2 * g
                pltpu.async_copy(table.at[src_v.at[j0 + 1]], buf1, sem1)
                pltpu.make_async_copy(table.at[src_v.at[j0]], buf0,
                                      sem0).wait()
                pltpu.sync_copy(buf0, acc.at[dst_v.at[j0]], add=True)
                pltpu.async_copy(table.at[src_v.at[(j0 + 2) % WIN]], buf0,
                                 sem0)
                pltpu.make_async_copy(table.at[src_v.at[j0 + 1]], buf1,
                                      sem1).wait()
                pltpu.sync_copy(buf1, acc.at[dst_v.at[j0 + 1]], add=True)

            pltpu.make_async_copy(table.at[src_v.at[0]], buf0, sem0).wait()

        plsc.subcore_barrier()

        pltpu.sync_copy(acc.at[pl.ds(base, ROWS_PER_TILE)],
                        out_hbm.at[cid].at[p].at[pl.ds(base, ROWS_PER_TILE)])


_sc_aggregate = functools.partial(
    pl.kernel,
    out_type=jax.ShapeDtypeStruct((NC, 2, ACC_ROWS, HD), jnp.float32),
    mesh=plsc.VectorSubcoreMesh(core_axis_name="c", subcore_axis_name="s",
                                num_cores=NC, num_subcores=NS),
    scratch_types=[
        pltpu.VMEM_SHARED((ACC_ROWS, HD), jnp.float32),
        pltpu.VMEM_SHARED((ACC_ROWS, HD), jnp.float32),
        pltpu.VMEM((WIN, CH), jnp.int32),
        pltpu.VMEM((WIN, CH), jnp.int32),
        pltpu.VMEM((2, CH, HD), jnp.float32),
        pltpu.VMEM((ZROWS, HD), jnp.float32),
        pltpu.SemaphoreType.DMA,
        pltpu.SemaphoreType.DMA,
    ],
)(_sc_aggregate_body)


def _tc_layer_body(a00_ref, a01_ref, a10_ref, a11_ref, w_ref, b_ref,
                   batch_ref, h2_ref, pool_ref):
    i = pl.program_id(0)
    agg = jnp.concatenate(
        [a00_ref[0, 0] + a10_ref[0, 0], a01_ref[0, 0] + a11_ref[0, 0]],
        axis=1)
    h = jnp.dot(agg, w_ref[...], preferred_element_type=jnp.float32)
    h = jnp.maximum(h + b_ref[...], 0.0)
    h2_ref[0, ...] = h[:, :HD]
    h2_ref[1, ...] = h[:, HD:]
    bt = batch_ref[0, 0, :]
    onehot = (bt[:, None] ==
              lax.broadcasted_iota(jnp.int32, (BLK, G), 1)).astype(jnp.float32)
    part = lax.dot_general(onehot, h, (((0,), (0,)), ((), ())),
                           preferred_element_type=jnp.float32)

    @pl.when(i == 0)
    def _():
        pool_ref[...] = jnp.zeros_like(pool_ref)

    pool_ref[...] += part


def _tc_layer(parts, w, b, batch3):
    return pl.pallas_call(
        _tc_layer_body,
        grid=(N // BLK,),
        in_specs=[
            pl.BlockSpec((1, 1, BLK, HD), lambda i: (0, 0, i, 0)),
            pl.BlockSpec((1, 1, BLK, HD), lambda i: (0, 1, i, 0)),
            pl.BlockSpec((1, 1, BLK, HD), lambda i: (1, 0, i, 0)),
            pl.BlockSpec((1, 1, BLK, HD), lambda i: (1, 1, i, 0)),
            pl.BlockSpec((D, D), lambda i: (0, 0)),
            pl.BlockSpec((1, D), lambda i: (0, 0)),
            pl.BlockSpec((1, 1, BLK), lambda i: (i, 0, 0)),
        ],
        out_specs=[
            pl.BlockSpec((2, BLK, HD), lambda i: (0, i, 0)),
            pl.BlockSpec((G, D), lambda i: (0, 0)),
        ],
        out_shape=[
            jax.ShapeDtypeStruct((2, ACC_ROWS, HD), jnp.float32),
            jax.ShapeDtypeStruct((G, D), jnp.float32),
        ],
    )(parts, parts, parts, parts, w, b, batch3)


def _head_body(p1_ref, p2_ref, p3_ref, wc_ref, bc_ref, out_ref):
    logits = (jnp.dot(p1_ref[...], wc_ref[0:D, :],
                      preferred_element_type=jnp.float32)
              + jnp.dot(p2_ref[...], wc_ref[D:2 * D, :],
                        preferred_element_type=jnp.float32)
              + jnp.dot(p3_ref[...], wc_ref[2 * D:3 * D, :],
                        preferred_element_type=jnp.float32))
    out_ref[...] = jax.nn.sigmoid(logits + bc_ref[0, 0])


def _head(p1, p2, p3, wc, bc):
    return pl.pallas_call(
        _head_body,
        out_shape=jax.ShapeDtypeStruct((G, 1), jnp.float32),
    )(p1, p2, p3, wc, bc)


def kernel(x, edge_index, batch, W1, b1, W2, b2, W3, b3, Wc, bc):
    src = edge_index[0]
    dst = edge_index[1]
    pad = EPAD - E
    pad_iota = jnp.arange(pad, dtype=jnp.int32)
    srcp = jnp.concatenate([src, pad_iota % N]).reshape(NW, CPW, CH)
    pad_dst = N + (pad_iota % (ACC_ROWS - N))
    dstp = jnp.concatenate([dst, pad_dst]).reshape(NW, CPW, CH)
    batch3 = batch.reshape(N // BLK, 1, BLK)

    h2 = jnp.pad(jnp.stack([x[:, :HD], x[:, HD:]]),
                 ((0, 0), (0, ACC_ROWS - N), (0, 0)))

    pooled = []
    for W, b in ((W1, b1), (W2, b2), (W3, b3)):
        parts = _sc_aggregate(h2, srcp, dstp)
        h2, pool = _tc_layer(parts, W, b.reshape(1, D), batch3)
        pooled.append(pool)

    out = _head(pooled[0], pooled[1], pooled[2], Wc, bc.reshape(1, 1))
    return out.reshape(-1)


# speedup vs baseline: 1.2254x; 1.2254x over previous
import functools

import jax
import jax.numpy as jnp
from jax import lax
from jax.experimental import pallas as pl
from jax.experimental.pallas import tpu as pltpu
from jax.experimental.pallas import tpu_sc as plsc

N = 10000
E = 320000
D = 128
G = 64

NC = 2
NS = 16
NW = NC * NS

CH = 128
CPW = 80
EPAD = NW * CPW * CH
WIN = 16
NWIN = CPW // WIN

ACC_ROWS = 10240
ZROWS = 64
ROWS_PER_TILE = ACC_ROWS // NS

BLK = 2000


def _sc_aggregate_body(h_hbm, src_hbm, dst_hbm, out_hbm,
                       acc, src_v, dst_v, rows_v, zbuf,
                       sem0, sem1, sem2, sem3):
    cid = lax.axis_index("c")
    sid = lax.axis_index("s")
    wid = cid * NS + sid

    zeros16 = jnp.zeros((16,), jnp.float32)

    @pl.loop(0, ZROWS)
    def _zrow(i):
        @pl.loop(0, D // 16)
        def _zcol(j):
            zbuf[i, pl.ds(j * 16, 16)] = zeros16

    @pl.loop(0, ACC_ROWS // NS // ZROWS)
    def _zacc(r):
        pltpu.sync_copy(zbuf, acc.at[pl.ds(sid * (ACC_ROWS // NS) + r * ZROWS,
                                           ZROWS)])

    plsc.subcore_barrier()

    buf0 = rows_v.at[0]
    buf1 = rows_v.at[1]

    @pl.loop(0, NWIN)
    def _win(w):
        pltpu.sync_copy(src_hbm.at[wid].at[pl.ds(w * WIN, WIN)], src_v)
        pltpu.sync_copy(dst_hbm.at[wid].at[pl.ds(w * WIN, WIN)], dst_v)
        pltpu.async_copy(h_hbm.at[src_v.at[0]], buf0, sem0)
        pltpu.async_copy(h_hbm.at[src_v.at[1]], buf1, sem1)

        @pl.loop(0, WIN // 2)
        def _pair(g):
            j0 = 2 * g
            pltpu.make_async_copy(h_hbm.at[src_v.at[j0]], buf0, sem0).wait()
            pltpu.async_copy(buf0, acc.at[dst_v.at[j0]], sem2, add=True)
            pltpu.make_async_copy(h_hbm.at[src_v.at[j0 + 1]], buf1,
                                  sem1).wait()
            pltpu.async_copy(buf1, acc.at[dst_v.at[j0 + 1]], sem3, add=True)
            pltpu.make_async_copy(buf0, acc.at[dst_v.at[j0]], sem2).wait()
            pltpu.async_copy(h_hbm.at[src_v.at[(j0 + 2) % WIN]], buf0, sem0)
            pltpu.make_async_copy(buf1, acc.at[dst_v.at[j0 + 1]],
                                  sem3).wait()
            pltpu.async_copy(h_hbm.at[src_v.at[(j0 + 3) % WIN]], buf1, sem1)

        pltpu.make_async_copy(h_hbm.at[src_v.at[0]], buf0, sem0).wait()
        pltpu.make_async_copy(h_hbm.at[src_v.at[1]], buf1, sem1).wait()

    plsc.subcore_barrier()

    pltpu.sync_copy(acc.at[pl.ds(sid * ROWS_PER_TILE, ROWS_PER_TILE)],
                    out_hbm.at[cid].at[pl.ds(sid * ROWS_PER_TILE,
                                             ROWS_PER_TILE)])


_sc_aggregate = functools.partial(
    pl.kernel,
    out_type=jax.ShapeDtypeStruct((NC, ACC_ROWS, D), jnp.float32),
    mesh=plsc.VectorSubcoreMesh(core_axis_name="c", subcore_axis_name="s",
                                num_cores=NC, num_subcores=NS),
    scratch_types=[
        pltpu.VMEM_SHARED((ACC_ROWS, D), jnp.float32),
        pltpu.VMEM((WIN, CH), jnp.int32),
        pltpu.VMEM((WIN, CH), jnp.int32),
        pltpu.VMEM((2, CH, D), jnp.float32),
        pltpu.VMEM((ZROWS, D), jnp.float32),
        pltpu.SemaphoreType.DMA,
        pltpu.SemaphoreType.DMA,
        pltpu.SemaphoreType.DMA,
        pltpu.SemaphoreType.DMA,
    ],
)(_sc_aggregate_body)


def _tc_layer_body(a0_ref, a1_ref, w_ref, b_ref, batch_ref, h_ref, pool_ref):
    i = pl.program_id(0)
    agg = a0_ref[0] + a1_ref[0]
    h = jnp.dot(agg, w_ref[...], preferred_element_type=jnp.float32)
    h = jnp.maximum(h + b_ref[...], 0.0)
    h_ref[...] = h
    bt = batch_ref[0, 0, :]
    onehot = (bt[:, None] ==
              lax.broadcasted_iota(jnp.int32, (BLK, G), 1)).astype(jnp.float32)
    part = lax.dot_general(onehot, h, (((0,), (0,)), ((), ())),
                           preferred_element_type=jnp.float32)

    @pl.when(i == 0)
    def _():
        pool_ref[...] = jnp.zeros_like(pool_ref)

    pool_ref[...] += part


def _tc_layer(parts, w, b, batch3):
    return pl.pallas_call(
        _tc_layer_body,
        grid=(N // BLK,),
        in_specs=[
            pl.BlockSpec((1, BLK, D), lambda i: (0, i, 0)),
            pl.BlockSpec((1, BLK, D), lambda i: (1, i, 0)),
            pl.BlockSpec((D, D), lambda i: (0, 0)),
            pl.BlockSpec((1, D), lambda i: (0, 0)),
            pl.BlockSpec((1, 1, BLK), lambda i: (i, 0, 0)),
        ],
        out_specs=[
            pl.BlockSpec((BLK, D), lambda i: (i, 0)),
            pl.BlockSpec((G, D), lambda i: (0, 0)),
        ],
        out_shape=[
            jax.ShapeDtypeStruct((N, D), jnp.float32),
            jax.ShapeDtypeStruct((G, D), jnp.float32),
        ],
    )(parts, parts, w, b, batch3)


def _head_body(p1_ref, p2_ref, p3_ref, wc_ref, bc_ref, out_ref):
    logits = (jnp.dot(p1_ref[...], wc_ref[0:D, :],
                      preferred_element_type=jnp.float32)
              + jnp.dot(p2_ref[...], wc_ref[D:2 * D, :],
                        preferred_element_type=jnp.float32)
              + jnp.dot(p3_ref[...], wc_ref[2 * D:3 * D, :],
                        preferred_element_type=jnp.float32))
    out_ref[...] = jax.nn.sigmoid(logits + bc_ref[0, 0])


def _head(p1, p2, p3, wc, bc):
    return pl.pallas_call(
        _head_body,
        out_shape=jax.ShapeDtypeStruct((G, 1), jnp.float32),
    )(p1, p2, p3, wc, bc)


def kernel(x, edge_index, batch, W1, b1, W2, b2, W3, b3, Wc, bc):
    src = edge_index[0]
    dst = edge_index[1]
    pad = EPAD - E
    pad_iota = jnp.arange(pad, dtype=jnp.int32)
    srcp = jnp.concatenate([src, pad_iota % N]).reshape(NW, CPW, CH)
    pad_dst = N + (pad_iota % (ACC_ROWS - N))
    dstp = jnp.concatenate([dst, pad_dst]).reshape(NW, CPW, CH)
    batch3 = batch.reshape(N // BLK, 1, BLK)

    h = x
    pooled = []
    for W, b in ((W1, b1), (W2, b2), (W3, b3)):
        parts = _sc_aggregate(h, srcp, dstp)
        h, pool = _tc_layer(parts, W, b.reshape(1, D), batch3)
        pooled.append(pool)

    out = _head(pooled[0], pooled[1], pooled[2], Wc, bc.reshape(1, 1))
    return out.reshape(-1)


# speedup vs baseline: 1.6512x; 1.3475x over previous
import functools

import jax
import jax.numpy as jnp
from jax import lax
from jax.experimental import pallas as pl
from jax.experimental.pallas import tpu as pltpu
from jax.experimental.pallas import tpu_sc as plsc

N = 10000
E = 320000
D = 128
G = 64

NC = 2
NS = 16
NW = NC * NS

CH = 128
CPW = 80
EPAD = NW * CPW * CH
WIN = 16
NWIN = CPW // WIN

ACC_ROWS = 10240
ZROWS = 64
ROWS_PER_TILE = ACC_ROWS // NS

BLK = 2000


def _sc_aggregate_body(h_hbm, ei_hbm, out_hbm,
                       acc, src_v, dst_v, rows_v, zbuf, sem0, sem1):
    cid = lax.axis_index("c")
    sid = lax.axis_index("s")
    wid = cid * NS + sid
    buf0 = rows_v.at[0]
    buf1 = rows_v.at[1]

    pltpu.sync_copy(ei_hbm.at[0].at[wid].at[pl.ds(0, WIN)], src_v)
    pltpu.sync_copy(ei_hbm.at[1].at[wid].at[pl.ds(0, WIN)], dst_v)
    pltpu.async_copy(h_hbm.at[src_v.at[0]], buf0, sem0)

    zeros16 = jnp.zeros((16,), jnp.float32)

    @pl.loop(0, ZROWS)
    def _zrow(i):
        @pl.loop(0, D // 16)
        def _zcol(j):
            zbuf[i, pl.ds(j * 16, 16)] = zeros16

    @pl.loop(0, ACC_ROWS // NS // ZROWS)
    def _zacc(r):
        pltpu.sync_copy(zbuf, acc.at[pl.ds(sid * (ACC_ROWS // NS) + r * ZROWS,
                                           ZROWS)])

    plsc.subcore_barrier()

    @pl.loop(0, NWIN)
    def _win(w):
        @pl.when(w > 0)
        def _stage():
            pltpu.sync_copy(ei_hbm.at[0].at[wid].at[pl.ds(w * WIN, WIN)],
                            src_v)
            pltpu.sync_copy(ei_hbm.at[1].at[wid].at[pl.ds(w * WIN, WIN)],
                            dst_v)
            pltpu.async_copy(h_hbm.at[src_v.at[0]], buf0, sem0)

        @pl.loop(0, WIN // 2)
        def _pair(g):
            j0 = 2 * g
            pltpu.async_copy(h_hbm.at[src_v.at[j0 + 1]], buf1, sem1)
            pltpu.make_async_copy(h_hbm.at[src_v.at[j0]], buf0, sem0).wait()
            pltpu.sync_copy(buf0, acc.at[dst_v.at[j0]], add=True)

            @pl.when(j0 + 2 < WIN)
            def _refill():
                pltpu.async_copy(h_hbm.at[src_v.at[j0 + 2]], buf0, sem0)

            pltpu.make_async_copy(h_hbm.at[src_v.at[j0 + 1]], buf1,
                                  sem1).wait()
            pltpu.sync_copy(buf1, acc.at[dst_v.at[j0 + 1]], add=True)

    plsc.subcore_barrier()

    pltpu.sync_copy(acc.at[pl.ds(sid * ROWS_PER_TILE, ROWS_PER_TILE)],
                    out_hbm.at[cid].at[pl.ds(sid * ROWS_PER_TILE,
                                             ROWS_PER_TILE)])


_sc_aggregate = functools.partial(
    pl.kernel,
    out_type=jax.ShapeDtypeStruct((NC, ACC_ROWS, D), jnp.float32),
    mesh=plsc.VectorSubcoreMesh(core_axis_name="c", subcore_axis_name="s",
                                num_cores=NC, num_subcores=NS),
    scratch_types=[
        pltpu.VMEM_SHARED((ACC_ROWS, D), jnp.float32),
        pltpu.VMEM((WIN, CH), jnp.int32),
        pltpu.VMEM((WIN, CH), jnp.int32),
        pltpu.VMEM((2, CH, D), jnp.float32),
        pltpu.VMEM((ZROWS, D), jnp.float32),
        pltpu.SemaphoreType.DMA,
        pltpu.SemaphoreType.DMA,
    ],
)(_sc_aggregate_body)


def _tc_layer_body(a0_ref, a1_ref, w_ref, b_ref, batch_ref, h_ref, pool_ref):
    i = pl.program_id(0)
    agg = a0_ref[0] + a1_ref[0]
    h = jnp.dot(agg, w_ref[...], preferred_element_type=jnp.float32)
    h = jnp.maximum(h + b_ref[...], 0.0)
    h_ref[...] = h
    bt = batch_ref[0, 0, :]
    onehot = (bt[:, None] ==
              lax.broadcasted_iota(jnp.int32, (BLK, G), 1)).astype(jnp.float32)
    part = lax.dot_general(onehot, h, (((0,), (0,)), ((), ())),
                           preferred_element_type=jnp.float32)

    @pl.when(i == 0)
    def _():
        pool_ref[...] = jnp.zeros_like(pool_ref)

    pool_ref[...] += part


def _tc_layer(parts, w, b, batch3):
    return pl.pallas_call(
        _tc_layer_body,
        grid=(N // BLK,),
        in_specs=[
            pl.BlockSpec((1, BLK, D), lambda i: (0, i, 0)),
            pl.BlockSpec((1, BLK, D), lambda i: (1, i, 0)),
            pl.BlockSpec((D, D), lambda i: (0, 0)),
            pl.BlockSpec((1, D), lambda i: (0, 0)),
            pl.BlockSpec((1, 1, BLK), lambda i: (i, 0, 0)),
        ],
        out_specs=[
            pl.BlockSpec((BLK, D), lambda i: (i, 0)),
            pl.BlockSpec((G, D), lambda i: (0, 0)),
        ],
        out_shape=[
            jax.ShapeDtypeStruct((N, D), jnp.float32),
            jax.ShapeDtypeStruct((G, D), jnp.float32),
        ],
    )(parts, parts, w, b, batch3)


def _head_body(p1_ref, p2_ref, p3_ref, wc_ref, bc_ref, out_ref):
    logits = (jnp.dot(p1_ref[...], wc_ref[0:D, :],
                      preferred_element_type=jnp.float32)
              + jnp.dot(p2_ref[...], wc_ref[D:2 * D, :],
                        preferred_element_type=jnp.float32)
              + jnp.dot(p3_ref[...], wc_ref[2 * D:3 * D, :],
                        preferred_element_type=jnp.float32))
    out_ref[...] = jax.nn.sigmoid(logits + bc_ref[0, 0])


def _head(p1, p2, p3, wc, bc):
    return pl.pallas_call(
        _head_body,
        out_shape=jax.ShapeDtypeStruct((G, 1), jnp.float32),
    )(p1, p2, p3, wc, bc)


def kernel(x, edge_index, batch, W1, b1, W2, b2, W3, b3, Wc, bc):
    pad = EPAD - E
    pad_iota = jnp.arange(pad, dtype=jnp.int32)
    pads2 = jnp.stack([pad_iota % N, N + (pad_iota % (ACC_ROWS - N))])
    ei_p = jnp.concatenate([edge_index, pads2], axis=1).reshape(
        2, NW, CPW, CH)
    batch3 = batch.reshape(N // BLK, 1, BLK)

    h = x
    pooled = []
    for W, b in ((W1, b1), (W2, b2), (W3, b3)):
        parts = _sc_aggregate(h, ei_p)
        h, pool = _tc_layer(parts, W, b.reshape(1, D), batch3)
        pooled.append(pool)

    out = _head(pooled[0], pooled[1], pooled[2], Wc, bc.reshape(1, 1))
    return out.reshape(-1)


# speedup vs baseline: 1.7615x; 1.0668x over previous
import functools

import jax
import jax.numpy as jnp
from jax import lax
from jax.experimental import pallas as pl
from jax.experimental.pallas import tpu as pltpu
from jax.experimental.pallas import tpu_sc as plsc

N = 10000
E = 320000
D = 128
G = 64

NC = 2
NS = 16
NW = NC * NS

CH = 128
CPW = 80
EPAD = NW * CPW * CH
WIN = 40
NWIN = CPW // WIN

ACC_ROWS = 10240
ZROWS = 40
ROWS_PER_TILE = ACC_ROWS // NS

BLK = 2000


def _sc_aggregate_body(h_hbm, ei_hbm, out_hbm,
                       acc, src_v, dst_v, rows_v, zbuf, sem0, sem1):
    cid = lax.axis_index("c")
    sid = lax.axis_index("s")
    wid = cid * NS + sid
    buf0 = rows_v.at[0]
    buf1 = rows_v.at[1]

    pltpu.sync_copy(ei_hbm.at[0].at[wid].at[pl.ds(0, WIN)], src_v)
    pltpu.sync_copy(ei_hbm.at[1].at[wid].at[pl.ds(0, WIN)], dst_v)
    pltpu.async_copy(h_hbm.at[src_v.at[0]], buf0, sem0)

    zeros16 = jnp.zeros((16,), jnp.float32)

    @pl.loop(0, ZROWS)
    def _zrow(i):
        @pl.loop(0, D // 16)
        def _zcol(j):
            zbuf[i, pl.ds(j * 16, 16)] = zeros16

    @pl.loop(0, ACC_ROWS // NS // ZROWS)
    def _zacc(r):
        pltpu.sync_copy(zbuf, acc.at[pl.ds(sid * (ACC_ROWS // NS) + r * ZROWS,
                                           ZROWS)])

    plsc.subcore_barrier()

    @pl.loop(0, NWIN)
    def _win(w):
        @pl.when(w > 0)
        def _stage():
            pltpu.sync_copy(ei_hbm.at[0].at[wid].at[pl.ds(w * WIN, WIN)],
                            src_v)
            pltpu.sync_copy(ei_hbm.at[1].at[wid].at[pl.ds(w * WIN, WIN)],
                            dst_v)
            pltpu.async_copy(h_hbm.at[src_v.at[0]], buf0, sem0)

        @pl.loop(0, WIN // 2)
        def _pair(g):
            j0 = 2 * g
            pltpu.async_copy(h_hbm.at[src_v.at[j0 + 1]], buf1, sem1)
            pltpu.make_async_copy(h_hbm.at[src_v.at[j0]], buf0, sem0).wait()
            pltpu.sync_copy(buf0, acc.at[dst_v.at[j0]], add=True)

            @pl.when(j0 + 2 < WIN)
            def _refill():
                pltpu.async_copy(h_hbm.at[src_v.at[j0 + 2]], buf0, sem0)

            pltpu.make_async_copy(h_hbm.at[src_v.at[j0 + 1]], buf1,
                                  sem1).wait()
            pltpu.sync_copy(buf1, acc.at[dst_v.at[j0 + 1]], add=True)

    plsc.subcore_barrier()

    pltpu.sync_copy(acc.at[pl.ds(sid * ROWS_PER_TILE, ROWS_PER_TILE)],
                    out_hbm.at[cid].at[pl.ds(sid * ROWS_PER_TILE,
                                             ROWS_PER_TILE)])


_sc_aggregate = functools.partial(
    pl.kernel,
    out_type=jax.ShapeDtypeStruct((NC, ACC_ROWS, D), jnp.float32),
    mesh=plsc.VectorSubcoreMesh(core_axis_name="c", subcore_axis_name="s",
                                num_cores=NC, num_subcores=NS),
    scratch_types=[
        pltpu.VMEM_SHARED((ACC_ROWS, D), jnp.float32),
        pltpu.VMEM((WIN, CH), jnp.int32),
        pltpu.VMEM((WIN, CH), jnp.int32),
        pltpu.VMEM((2, CH, D), jnp.float32),
        pltpu.VMEM((ZROWS, D), jnp.float32),
        pltpu.SemaphoreType.DMA,
        pltpu.SemaphoreType.DMA,
    ],
)(_sc_aggregate_body)


def _tc_layer_body(a0_ref, a1_ref, w_ref, b_ref, batch_ref, h_ref, pool_ref):
    i = pl.program_id(0)
    agg = a0_ref[0] + a1_ref[0]
    h = jnp.dot(agg, w_ref[...], preferred_element_type=jnp.float32)
    h = jnp.maximum(h + b_ref[...], 0.0)
    h_ref[...] = h
    bt = batch_ref[0, 0, :]
    onehot = (bt[:, None] ==
              lax.broadcasted_iota(jnp.int32, (BLK, G), 1)).astype(jnp.float32)
    part = lax.dot_general(onehot, h, (((0,), (0,)), ((), ())),
                           preferred_element_type=jnp.float32)

    @pl.when(i == 0)
    def _():
        pool_ref[...] = jnp.zeros_like(pool_ref)

    pool_ref[...] += part


def _tc_layer(parts, w, b, batch3):
    return pl.pallas_call(
        _tc_layer_body,
        grid=(N // BLK,),
        in_specs=[
            pl.BlockSpec((1, BLK, D), lambda i: (0, i, 0)),
            pl.BlockSpec((1, BLK, D), lambda i: (1, i, 0)),
            pl.BlockSpec((D, D), lambda i: (0, 0)),
            pl.BlockSpec((1, D), lambda i: (0, 0)),
            pl.BlockSpec((1, 1, BLK), lambda i: (i, 0, 0)),
        ],
        out_specs=[
            pl.BlockSpec((BLK, D), lambda i: (i, 0)),
            pl.BlockSpec((G, D), lambda i: (0, 0)),
        ],
        out_shape=[
            jax.ShapeDtypeStruct((N, D), jnp.float32),
            jax.ShapeDtypeStruct((G, D), jnp.float32),
        ],
    )(parts, parts, w, b, batch3)


def _head_body(p1_ref, p2_ref, p3_ref, wc_ref, bc_ref, out_ref):
    logits = (jnp.dot(p1_ref[...], wc_ref[0:D, :],
                      preferred_element_type=jnp.float32)
              + jnp.dot(p2_ref[...], wc_ref[D:2 * D, :],
                        preferred_element_type=jnp.float32)
              + jnp.dot(p3_ref[...], wc_ref[2 * D:3 * D, :],
                        preferred_element_type=jnp.float32))
    out_ref[...] = jax.nn.sigmoid(logits + bc_ref[0, 0])


def _head(p1, p2, p3, wc, bc):
    return pl.pallas_call(
        _head_body,
        out_shape=jax.ShapeDtypeStruct((G, 1), jnp.float32),
    )(p1, p2, p3, wc, bc)


def kernel(x, edge_index, batch, W1, b1, W2, b2, W3, b3, Wc, bc):
    pad = EPAD - E
    pad_iota = jnp.arange(pad, dtype=jnp.int32)
    pads2 = jnp.stack([pad_iota % N, N + (pad_iota % (ACC_ROWS - N))])
    ei_p = jnp.concatenate([edge_index, pads2], axis=1).reshape(
        2, NW, CPW, CH)
    batch3 = batch.reshape(N // BLK, 1, BLK)

    h = x
    pooled = []
    for W, b in ((W1, b1), (W2, b2), (W3, b3)):
        parts = _sc_aggregate(h, ei_p)
        h, pool = _tc_layer(parts, W, b.reshape(1, D), batch3)
        pooled.append(pool)

    out = _head(pooled[0], pooled[1], pooled[2], Wc, bc.reshape(1, 1))
    return out.reshape(-1)
